# Initial kernel scaffold; baseline (speedup 1.0000x reference)
import functools

import jax
import jax.numpy as jnp
from jax import lax
from jax.experimental import pallas as pl
from jax.experimental.pallas import tpu as pltpu
from jax.experimental.pallas import tpu_sc as plsc

N_NODES = 10000
D = 128
E = 320000

NC = 2
NS = 16
NW = NC * NS

CHUNK = 80
EDGES_PER_TILE = E // NW
NCHUNK = EDGES_PER_TILE // CHUNK
ROWS_PER_TILE = N_NODES // NS
CW = 16


def _sc_accumulate(src_hbm, dst_hbm, x_hbm, ones_hbm, zf_hbm, zc_hbm,
                   partial_hbm, cnt_hbm,
                   src_v, dst_v, rows_v, ones_v, acc_sh, cnt_sh, sem):
    c = lax.axis_index("c")
    s = lax.axis_index("s")
    wid = s * NC + c
    row0 = s * ROWS_PER_TILE

    pltpu.sync_copy(zf_hbm, acc_sh.at[pl.ds(row0, ROWS_PER_TILE)])
    pltpu.sync_copy(zc_hbm, cnt_sh.at[pl.ds(row0, ROWS_PER_TILE)])
    pltpu.sync_copy(src_hbm.at[wid], src_v)
    pltpu.sync_copy(dst_hbm.at[wid], dst_v)
    pltpu.sync_copy(ones_hbm, ones_v)
    plsc.subcore_barrier()

    def body(j, carry):
        pltpu.async_copy(x_hbm.at[src_v.at[j]], rows_v, sem).wait()
        pltpu.sync_copy(rows_v, acc_sh.at[dst_v.at[j]], add=True)
        pltpu.sync_copy(ones_v, cnt_sh.at[dst_v.at[j]], add=True)
        return carry

    lax.fori_loop(0, NCHUNK, body, 0)
    plsc.subcore_barrier()

    pltpu.sync_copy(acc_sh.at[pl.ds(row0, ROWS_PER_TILE)],
                    partial_hbm.at[c, pl.ds(row0, ROWS_PER_TILE)])
    pltpu.sync_copy(cnt_sh.at[pl.ds(row0, ROWS_PER_TILE)],
                    cnt_hbm.at[c, pl.ds(row0, ROWS_PER_TILE)])


def _combine_body(p_ref, c_ref, o_ref):
    p = p_ref[0] + p_ref[1]
    cnt = c_ref[0, :, 0] + c_ref[1, :, 0]
    o_ref[...] = p * (1.0 / jnp.maximum(cnt, 1.0))[:, None]


def kernel(x, edge_index):
    x = x.astype(jnp.float32)
    ei = edge_index.astype(jnp.int32)
    src = ei[0].reshape(NW, NCHUNK, CHUNK)
    dst = ei[1].reshape(NW, NCHUNK, CHUNK)

    ones = jnp.ones((CHUNK, CW), jnp.float32)
    zf = jnp.zeros((ROWS_PER_TILE, D), jnp.float32)
    zc = jnp.zeros((ROWS_PER_TILE, CW), jnp.float32)

    mesh = plsc.VectorSubcoreMesh(core_axis_name="c", subcore_axis_name="s")
    sc_fn = functools.partial(
        pl.kernel,
        mesh=mesh,
        out_type=[
            jax.ShapeDtypeStruct((NC, N_NODES, D), jnp.float32),
            jax.ShapeDtypeStruct((NC, N_NODES, CW), jnp.float32),
        ],
        scratch_types=[
            pltpu.VMEM((NCHUNK, CHUNK), jnp.int32),
            pltpu.VMEM((NCHUNK, CHUNK), jnp.int32),
            pltpu.VMEM((CHUNK, D), jnp.float32),
            pltpu.VMEM((CHUNK, CW), jnp.float32),
            pltpu.VMEM_SHARED((N_NODES, D), jnp.float32),
            pltpu.VMEM_SHARED((N_NODES, CW), jnp.float32),
            pltpu.SemaphoreType.DMA,
        ],
    )(_sc_accumulate)
    partial, cnt = sc_fn(src, dst, x, ones, zf, zc)

    BLK = 1000
    h = pl.pallas_call(
        _combine_body,
        grid=(N_NODES // BLK,),
        in_specs=[
            pl.BlockSpec((NC, BLK, D), lambda i: (0, i, 0)),
            pl.BlockSpec((NC, BLK, CW), lambda i: (0, i, 0)),
        ],
        out_specs=pl.BlockSpec((BLK, D), lambda i: (i, 0)),
        out_shape=jax.ShapeDtypeStruct((N_NODES, D), jnp.float32),
    )(partial, cnt)
    return h


# speedup vs baseline: 8.6960x; 8.6960x over previous
import functools

import jax
import jax.numpy as jnp
from jax import lax
from jax.experimental import pallas as pl
from jax.experimental.pallas import tpu as pltpu
from jax.experimental.pallas import tpu_sc as plsc

N_NODES = 10000
D = 128
E = 320000

NC = 2
NS = 16
NW = NC * NS
L = 16

CHUNK = 80
EDGES_PER_TILE = E // NW
NCHUNK = EDGES_PER_TILE // CHUNK
NPAD = 10240
ROWS_PER_TILE = NPAD // NS


def _sc_accumulate(src_hbm, dst_hbm, x_hbm, zf_hbm,
                   partial_hbm, cntw_hbm,
                   src_v, dst_v, rows_v, cnt_v, acc_sh, sem):
    c = lax.axis_index("c")
    s = lax.axis_index("s")
    wid = s * NC + c
    row0 = s * ROWS_PER_TILE

    pltpu.sync_copy(zf_hbm, acc_sh.at[pl.ds(row0, ROWS_PER_TILE)])
    zv = jnp.zeros((L,), jnp.float32)

    def zbody(k, carry):
        cnt_v[pl.ds(k * L, L)] = zv
        return carry

    lax.fori_loop(0, NPAD // L, zbody, 0)
    pltpu.sync_copy(src_hbm.at[wid], src_v)
    pltpu.sync_copy(dst_hbm.at[wid], dst_v)
    plsc.subcore_barrier()

    ones_v = jnp.ones((L,), jnp.float32)

    def body(j, carry):
        pltpu.async_copy(x_hbm.at[src_v.at[pl.ds(j * CHUNK, CHUNK)]], rows_v,
                         sem).wait()
        pltpu.sync_copy(rows_v, acc_sh.at[dst_v.at[j]], add=True)
        for k in range(CHUNK // L):
            dv = dst_v[j, pl.ds(k * L, L)]
            plsc.addupdate_scatter(cnt_v, [dv], ones_v)
        return carry

    lax.fori_loop(0, NCHUNK, body, 0)
    plsc.subcore_barrier()

    pltpu.sync_copy(acc_sh.at[pl.ds(row0, ROWS_PER_TILE)],
                    partial_hbm.at[c, pl.ds(row0, ROWS_PER_TILE)])
    pltpu.sync_copy(cnt_v, cntw_hbm.at[wid])


def _combine_body(p_ref, c_ref, o_ref):
    p = p_ref[0] + p_ref[1]
    cnt = jnp.sum(c_ref[...], axis=0)
    o_ref[...] = p * (1.0 / jnp.maximum(cnt, 1.0))[:, None]


def kernel(x, edge_index):
    x = x.astype(jnp.float32)
    ei = edge_index.astype(jnp.int32)
    src = ei[0].reshape(NW, EDGES_PER_TILE)
    dst = ei[1].reshape(NW, NCHUNK, CHUNK)

    zf = jnp.zeros((ROWS_PER_TILE, D), jnp.float32)

    mesh = plsc.VectorSubcoreMesh(core_axis_name="c", subcore_axis_name="s")
    sc_fn = functools.partial(
        pl.kernel,
        mesh=mesh,
        compiler_params=pltpu.CompilerParams(needs_layout_passes=False),
        out_type=[
            jax.ShapeDtypeStruct((NC, NPAD, D), jnp.float32),
            jax.ShapeDtypeStruct((NW, NPAD), jnp.float32),
        ],
        scratch_types=[
            pltpu.VMEM((EDGES_PER_TILE,), jnp.int32),
            pltpu.VMEM((NCHUNK, CHUNK), jnp.int32),
            pltpu.VMEM((CHUNK, D), jnp.float32),
            pltpu.VMEM((NPAD,), jnp.float32),
            pltpu.VMEM_SHARED((NPAD, D), jnp.float32),
            pltpu.SemaphoreType.DMA,
        ],
    )(_sc_accumulate)
    partial, cntw = sc_fn(src, dst, x, zf)

    BLK = 1024
    h = pl.pallas_call(
        _combine_body,
        grid=(NPAD // BLK,),
        in_specs=[
            pl.BlockSpec((NC, BLK, D), lambda i: (0, i, 0)),
            pl.BlockSpec((NW, BLK), lambda i: (0, i)),
        ],
        out_specs=pl.BlockSpec((BLK, D), lambda i: (i, 0)),
        out_shape=jax.ShapeDtypeStruct((NPAD, D), jnp.float32),
    )(partial, cntw)
    return h[:N_NODES]
